# Initial kernel scaffold; baseline (speedup 1.0000x reference)
import functools

import jax
import jax.numpy as jnp
from jax import lax
from jax.experimental import pallas as pl
from jax.experimental.pallas import tpu as pltpu
from jax.experimental.pallas import tpu_sc as plsc

NC = 2
NS = 16
NW = NC * NS
CSZ = 125
DE = 16


def _sc_edge_kernel(n_nodes, n_edges, interpret=False):
    nch = n_edges // (NW * CSZ)
    epw = nch * CSZ
    rpt = n_nodes // NS

    mesh = plsc.VectorSubcoreMesh(core_axis_name="c", subcore_axis_name="s")

    @functools.partial(
        pl.kernel,
        out_type=(
            jax.ShapeDtypeStruct((n_edges, DE), jnp.float32),
            jax.ShapeDtypeStruct((NC * n_nodes, DE), jnp.float32),
            jax.ShapeDtypeStruct((NC * n_nodes, DE), jnp.float32),
        ),
        mesh=mesh,
        scratch_types=[
            pltpu.VMEM((nch, CSZ), jnp.int32),
            pltpu.VMEM((nch, CSZ), jnp.int32),
            pltpu.VMEM((CSZ, DE), jnp.float32),
            pltpu.VMEM((CSZ, DE), jnp.float32),
            pltpu.VMEM((CSZ, DE), jnp.float32),
            pltpu.VMEM((CSZ, DE), jnp.float32),
            pltpu.VMEM((CSZ, DE), jnp.float32),
            pltpu.VMEM((n_nodes // NS, DE), jnp.float32),
            pltpu.VMEM_SHARED((n_nodes, DE), jnp.float32),
            pltpu.VMEM_SHARED((n_nodes, DE), jnp.float32),
            pltpu.SemaphoreType.DMA,
            pltpu.SemaphoreType.DMA,
        ],
        interpret=interpret,
    )
    def k(a_hbm, b_hbm, c_hbm, src_hbm, dst_hbm, ones_hbm, zeros_hbm,
          he_hbm, sums_hbm, cnts_hbm,
          srcv, dstv, a_v, b_v, c_v, he_v, ones_v, z_v,
          sums_sh, cnts_sh, sem_a, sem_b):
        cid = lax.axis_index("c")
        sid = lax.axis_index("s")
        wid = sid * NC + cid

        pltpu.sync_copy(ones_hbm, ones_v)
        pltpu.sync_copy(zeros_hbm, z_v)
        pltpu.sync_copy(src_hbm.at[pl.ds(wid * nch, nch)], srcv)
        pltpu.sync_copy(dst_hbm.at[pl.ds(wid * nch, nch)], dstv)
        pltpu.sync_copy(z_v, sums_sh.at[pl.ds(sid * rpt, rpt)])
        pltpu.sync_copy(z_v, cnts_sh.at[pl.ds(sid * rpt, rpt)])
        plsc.subcore_barrier()

        def chunk(j, carry):
            base = wid * epw + j * CSZ
            ga = pltpu.async_copy(a_hbm.at[srcv.at[j]], a_v, sem_a)
            gb = pltpu.async_copy(b_hbm.at[dstv.at[j]], b_v, sem_b)
            pltpu.sync_copy(c_hbm.at[pl.ds(base, CSZ)], c_v)
            ga.wait()
            gb.wait()

            def row(i, c2):
                he_v[i, :] = a_v[i, :] + b_v[i, :] + c_v[i, :]
                return c2

            lax.fori_loop(0, CSZ, row, 0, unroll=5)
            pltpu.sync_copy(he_v, he_hbm.at[pl.ds(base, CSZ)])
            pltpu.sync_copy(he_v, sums_sh.at[dstv.at[j]], add=True)
            pltpu.sync_copy(ones_v, cnts_sh.at[dstv.at[j]], add=True)
            return carry

        lax.fori_loop(0, nch, chunk, 0)

        plsc.subcore_barrier()
        out0 = cid * n_nodes + sid * rpt
        pltpu.sync_copy(sums_sh.at[pl.ds(sid * rpt, rpt)],
                        sums_hbm.at[pl.ds(out0, rpt)])
        pltpu.sync_copy(cnts_sh.at[pl.ds(sid * rpt, rpt)],
                        cnts_hbm.at[pl.ds(out0, rpt)])

    return k


def _tc_pre(x, edge_attr, w12, w3, b_e):
    def ab_body(x_ref, w_ref, o_ref):
        o_ref[...] = jnp.dot(x_ref[...], w_ref[...],
                             preferred_element_type=jnp.float32)

    ab = pl.pallas_call(
        ab_body,
        out_shape=jax.ShapeDtypeStruct((x.shape[0], 2 * DE), jnp.float32),
    )(x, w12)

    n_edges = edge_attr.shape[0]
    grid = 8
    blk = n_edges // grid

    def c_body(ea_ref, w_ref, b_ref, o_ref):
        o_ref[...] = jnp.dot(ea_ref[...], w_ref[...],
                             preferred_element_type=jnp.float32) + b_ref[...]

    c = pl.pallas_call(
        c_body,
        grid=(grid,),
        in_specs=[
            pl.BlockSpec((blk, DE), lambda i: (i, 0)),
            pl.BlockSpec((DE, DE), lambda i: (0, 0)),
            pl.BlockSpec((1, DE), lambda i: (0, 0)),
        ],
        out_specs=pl.BlockSpec((blk, DE), lambda i: (i, 0)),
        out_shape=jax.ShapeDtypeStruct((n_edges, DE), jnp.float32),
    )(edge_attr, w3, b_e)
    return ab, c


def _tc_post(x, sums_p, cnts_p, wn_x, wn_a, b_n):
    def body(x_ref, s_ref, c_ref, wx_ref, wa_ref, b_ref, o_ref):
        sums = s_ref[0] + s_ref[1]
        cnts = jnp.maximum(c_ref[0] + c_ref[1], 1.0)
        agg = sums / cnts
        o_ref[...] = (
            jnp.dot(x_ref[...], wx_ref[...], preferred_element_type=jnp.float32)
            + jnp.dot(agg, wa_ref[...], preferred_element_type=jnp.float32)
            + b_ref[...]
        )

    return pl.pallas_call(
        body,
        out_shape=jax.ShapeDtypeStruct((x.shape[0], x.shape[1]), jnp.float32),
    )(x, sums_p, cnts_p, wn_x, wn_a, b_n)


def kernel(x, edge_index, edge_attr, W_edge, b_edge, W_node, b_node):
    n_nodes, d_node = x.shape
    n_edges = edge_index.shape[1]

    w12 = W_edge[: 2 * d_node]
    w12 = jnp.concatenate([w12[:d_node], w12[d_node:]], axis=1)
    w3 = W_edge[2 * d_node:]

    ab, c = _tc_pre(x, edge_attr, w12, w3, b_edge.reshape(1, DE))
    a = ab[:, :DE]
    b = ab[:, DE:]

    src = edge_index[0].reshape(n_edges // CSZ, CSZ)
    dst = edge_index[1].reshape(n_edges // CSZ, CSZ)
    ones = jnp.ones((CSZ, DE), jnp.float32)
    zeros = jnp.zeros((n_nodes // NS, DE), jnp.float32)

    h_e, sums_p, cnts_p = _sc_edge_kernel(n_nodes, n_edges)(
        a, b, c, src, dst, ones, zeros)

    h_v = _tc_post(
        x,
        sums_p.reshape(NC, n_nodes, DE),
        cnts_p.reshape(NC, n_nodes, DE),
        W_node[:d_node],
        W_node[d_node:],
        b_node.reshape(1, d_node),
    )
    return (h_v, edge_index, h_e)


# speedup vs baseline: 5.7124x; 5.7124x over previous
import functools

import jax
import jax.numpy as jnp
from jax import lax
from jax.experimental import pallas as pl
from jax.experimental.pallas import tpu as pltpu
from jax.experimental.pallas import tpu_sc as plsc

NC = 2
NS = 16
NW = NC * NS
CSZ = 80
DE = 16


def _acc_pad(n_nodes):
    rpt = -(-(n_nodes // NS) // 8) * 8
    return rpt, rpt * NS


def _sc_edge_kernel(n_nodes, n_edges, interpret=False):
    nch = n_edges // (NW * CSZ)
    epw = nch * CSZ
    rpt, n_pad = _acc_pad(n_nodes)

    mesh = plsc.VectorSubcoreMesh(core_axis_name="c", subcore_axis_name="s",
                                  num_cores=NC, num_subcores=NS)

    @functools.partial(
        pl.kernel,
        out_type=(
            jax.ShapeDtypeStruct((n_edges, DE), jnp.float32),
            jax.ShapeDtypeStruct((NC * n_pad, DE), jnp.float32),
            jax.ShapeDtypeStruct((NC * n_pad, DE), jnp.float32),
        ),
        mesh=mesh,
        scratch_types=[
            pltpu.VMEM((nch, CSZ), jnp.int32),
            pltpu.VMEM((nch, CSZ), jnp.int32),
            pltpu.VMEM((CSZ, DE), jnp.float32),
            pltpu.VMEM((CSZ, DE), jnp.float32),
            pltpu.VMEM((CSZ, DE), jnp.float32),
            pltpu.VMEM((CSZ, DE), jnp.float32),
            pltpu.VMEM((CSZ, DE), jnp.float32),
            pltpu.VMEM((rpt, DE), jnp.float32),
            pltpu.VMEM_SHARED((n_pad, DE), jnp.float32),
            pltpu.VMEM_SHARED((n_pad, DE), jnp.float32),
            pltpu.SemaphoreType.DMA,
            pltpu.SemaphoreType.DMA,
        ],
        compiler_params=pltpu.CompilerParams(use_tc_tiling_on_sc=False),
        interpret=interpret,
    )
    def k(a_hbm, b_hbm, c_hbm, src_hbm, dst_hbm, ones_hbm, zeros_hbm,
          he_hbm, sums_hbm, cnts_hbm,
          srcv, dstv, a_v, b_v, c_v, he_v, ones_v, z_v,
          sums_sh, cnts_sh, sem_a, sem_b):
        cid = lax.axis_index("c")
        sid = lax.axis_index("s")
        wid = sid * NC + cid

        pltpu.sync_copy(ones_hbm, ones_v)
        pltpu.sync_copy(zeros_hbm, z_v)
        pltpu.sync_copy(src_hbm.at[wid], srcv)
        pltpu.sync_copy(dst_hbm.at[wid], dstv)
        pltpu.sync_copy(z_v, sums_sh.at[pl.ds(sid * rpt, rpt)])
        pltpu.sync_copy(z_v, cnts_sh.at[pl.ds(sid * rpt, rpt)])
        plsc.subcore_barrier()

        def chunk(j, carry):
            base = wid * epw + j * CSZ
            ga = pltpu.async_copy(a_hbm.at[srcv.at[j]], a_v, sem_a)
            gb = pltpu.async_copy(b_hbm.at[dstv.at[j]], b_v, sem_b)
            pltpu.sync_copy(c_hbm.at[pl.ds(base, CSZ)], c_v)
            ga.wait()
            gb.wait()

            def row(i, c2):
                he_v[i, :] = a_v[i, :] + b_v[i, :] + c_v[i, :]
                return c2

            lax.fori_loop(0, CSZ, row, 0, unroll=5)
            pltpu.sync_copy(he_v, he_hbm.at[pl.ds(base, CSZ)])
            pltpu.sync_copy(he_v, sums_sh.at[dstv.at[j]], add=True)
            pltpu.sync_copy(ones_v, cnts_sh.at[dstv.at[j]], add=True)
            return carry

        lax.fori_loop(0, nch, chunk, 0)

        plsc.subcore_barrier()
        out0 = cid * n_pad + sid * rpt
        pltpu.sync_copy(sums_sh.at[pl.ds(sid * rpt, rpt)],
                        sums_hbm.at[pl.ds(out0, rpt)])
        pltpu.sync_copy(cnts_sh.at[pl.ds(sid * rpt, rpt)],
                        cnts_hbm.at[pl.ds(out0, rpt)])

    return k


def _tc_pre(x, edge_attr, w12, w3, b_e):
    def ab_body(x_ref, w_ref, o_ref):
        o_ref[...] = jnp.dot(x_ref[...], w_ref[...],
                             preferred_element_type=jnp.float32)

    ab = pl.pallas_call(
        ab_body,
        out_shape=jax.ShapeDtypeStruct((x.shape[0], 2 * DE), jnp.float32),
    )(x, w12)

    n_edges = edge_attr.shape[0]
    pack = 128 // DE
    ea8 = edge_attr.reshape(n_edges // pack, pack * DE)
    w3b = jnp.kron(jnp.eye(pack, dtype=jnp.float32), w3)
    b8 = jnp.tile(b_e, (1, pack))
    grid = 8
    blk = ea8.shape[0] // grid

    def c_body(ea_ref, w_ref, b_ref, o_ref):
        o_ref[...] = jnp.dot(ea_ref[...], w_ref[...],
                             preferred_element_type=jnp.float32) + b_ref[...]

    c8 = pl.pallas_call(
        c_body,
        grid=(grid,),
        in_specs=[
            pl.BlockSpec((blk, pack * DE), lambda i: (i, 0)),
            pl.BlockSpec((pack * DE, pack * DE), lambda i: (0, 0)),
            pl.BlockSpec((1, pack * DE), lambda i: (0, 0)),
        ],
        out_specs=pl.BlockSpec((blk, pack * DE), lambda i: (i, 0)),
        out_shape=jax.ShapeDtypeStruct(ea8.shape, jnp.float32),
    )(ea8, w3b, b8)
    return ab, c8.reshape(n_edges, DE)


def _tc_post(x, sums_p, cnts_p, wn_x, wn_a, b_n):
    def body(x_ref, s_ref, c_ref, wx_ref, wa_ref, b_ref, o_ref):
        sums = s_ref[0] + s_ref[1]
        cnts = jnp.maximum(c_ref[0] + c_ref[1], 1.0)
        agg = sums / cnts
        o_ref[...] = (
            jnp.dot(x_ref[...], wx_ref[...], preferred_element_type=jnp.float32)
            + jnp.dot(agg, wa_ref[...], preferred_element_type=jnp.float32)
            + b_ref[...]
        )

    return pl.pallas_call(
        body,
        out_shape=jax.ShapeDtypeStruct((x.shape[0], x.shape[1]), jnp.float32),
    )(x, sums_p, cnts_p, wn_x, wn_a, b_n)


def kernel(x, edge_index, edge_attr, W_edge, b_edge, W_node, b_node):
    n_nodes, d_node = x.shape
    n_edges = edge_index.shape[1]

    w12 = W_edge[: 2 * d_node]
    w12 = jnp.concatenate([w12[:d_node], w12[d_node:]], axis=1)
    w3 = W_edge[2 * d_node:]

    ab, c = _tc_pre(x, edge_attr, w12, w3, b_edge.reshape(1, DE))
    a = ab[:, :DE]
    b = ab[:, DE:]

    nch = n_edges // (NW * CSZ)
    rpt, n_pad = _acc_pad(n_nodes)
    src = edge_index[0].reshape(NW, nch, CSZ)
    dst = edge_index[1].reshape(NW, nch, CSZ)
    ones = jnp.ones((CSZ, DE), jnp.float32)
    zeros = jnp.zeros((rpt, DE), jnp.float32)

    h_e, sums_p, cnts_p = _sc_edge_kernel(n_nodes, n_edges)(
        a, b, c, src, dst, ones, zeros)

    h_v = _tc_post(
        x,
        sums_p.reshape(NC, n_pad, DE)[:, :n_nodes],
        cnts_p.reshape(NC, n_pad, DE)[:, :n_nodes],
        W_node[:d_node],
        W_node[d_node:],
        b_node.reshape(1, d_node),
    )
    return (h_v, edge_index, h_e)


# speedup vs baseline: 5.8276x; 1.0202x over previous
import functools

import jax
import jax.numpy as jnp
from jax import lax
from jax.experimental import pallas as pl
from jax.experimental.pallas import tpu as pltpu
from jax.experimental.pallas import tpu_sc as plsc

NC = 2
NS = 16
NW = NC * NS
CSZ = 80
DE = 16


def _acc_pad(n_nodes):
    rpt = -(-(n_nodes // NS) // 8) * 8
    return rpt, rpt * NS


def _sc_edge_kernel(n_nodes, n_edges, interpret=False):
    nch = n_edges // (NW * CSZ)
    epw = nch * CSZ
    rpt, n_pad = _acc_pad(n_nodes)

    mesh = plsc.VectorSubcoreMesh(core_axis_name="c", subcore_axis_name="s",
                                  num_cores=NC, num_subcores=NS)

    @functools.partial(
        pl.kernel,
        out_type=(
            jax.ShapeDtypeStruct((n_edges, DE), jnp.float32),
            jax.ShapeDtypeStruct((NC * n_pad, DE), jnp.float32),
            jax.ShapeDtypeStruct((NC * n_pad, DE), jnp.float32),
        ),
        mesh=mesh,
        scratch_types=[
            pltpu.VMEM((nch, CSZ), jnp.int32),
            pltpu.VMEM((nch, CSZ), jnp.int32),
            pltpu.VMEM((CSZ, DE), jnp.float32),
            pltpu.VMEM((CSZ, DE), jnp.float32),
            pltpu.VMEM((CSZ, DE), jnp.float32),
            pltpu.VMEM((CSZ, DE), jnp.float32),
            pltpu.VMEM((CSZ, DE), jnp.float32),
            pltpu.VMEM((rpt, DE), jnp.float32),
            pltpu.VMEM_SHARED((n_pad, DE), jnp.float32),
            pltpu.VMEM_SHARED((n_pad, DE), jnp.float32),
            pltpu.SemaphoreType.DMA,
            pltpu.SemaphoreType.DMA,
        ],
        compiler_params=pltpu.CompilerParams(use_tc_tiling_on_sc=False),
        interpret=interpret,
    )
    def k(a_hbm, b_hbm, c_hbm, src_hbm, dst_hbm, ones_hbm, zeros_hbm,
          he_hbm, sums_hbm, cnts_hbm,
          srcv, dstv, a_v, b_v, c_v, he_v, ones_v, z_v,
          sums_sh, cnts_sh, sem_a, sem_b):
        cid = lax.axis_index("c")
        sid = lax.axis_index("s")
        wid = sid * NC + cid

        pltpu.sync_copy(ones_hbm, ones_v)
        pltpu.sync_copy(zeros_hbm, z_v)
        pltpu.sync_copy(src_hbm.at[wid], srcv)
        pltpu.sync_copy(dst_hbm.at[wid], dstv)
        pltpu.sync_copy(z_v, sums_sh.at[pl.ds(sid * rpt, rpt)])
        pltpu.sync_copy(z_v, cnts_sh.at[pl.ds(sid * rpt, rpt)])
        plsc.subcore_barrier()

        def chunk(j, carry):
            base = wid * epw + j * CSZ
            ga = pltpu.async_copy(a_hbm.at[srcv.at[j]], a_v, sem_a)
            gb = pltpu.async_copy(b_hbm.at[dstv.at[j]], b_v, sem_b)
            pltpu.sync_copy(c_hbm.at[pl.ds(base, CSZ)], c_v)
            ga.wait()
            gb.wait()

            def row(i, c2):
                he_v[i, :] = a_v[i, :] + b_v[i, :] + c_v[i, :]
                return c2

            lax.fori_loop(0, CSZ, row, 0, unroll=5)
            pltpu.sync_copy(he_v, he_hbm.at[pl.ds(base, CSZ)])
            pltpu.sync_copy(he_v, sums_sh.at[dstv.at[j]], add=True)
            pltpu.sync_copy(ones_v, cnts_sh.at[dstv.at[j]], add=True)
            return carry

        lax.fori_loop(0, nch, chunk, 0)

        plsc.subcore_barrier()
        out0 = cid * n_pad + sid * rpt
        pltpu.sync_copy(sums_sh.at[pl.ds(sid * rpt, rpt)],
                        sums_hbm.at[pl.ds(out0, rpt)])
        pltpu.sync_copy(cnts_sh.at[pl.ds(sid * rpt, rpt)],
                        cnts_hbm.at[pl.ds(out0, rpt)])

    return k


def _tc_pre(x, edge_attr, w12, w3, b_e):
    def ab_body(x_ref, w_ref, oa_ref, ob_ref):
        r = jnp.dot(x_ref[...], w_ref[...], preferred_element_type=jnp.float32)
        oa_ref[...] = r[:, :DE]
        ob_ref[...] = r[:, DE:]

    a, b = pl.pallas_call(
        ab_body,
        out_shape=(
            jax.ShapeDtypeStruct((x.shape[0], DE), jnp.float32),
            jax.ShapeDtypeStruct((x.shape[0], DE), jnp.float32),
        ),
    )(x, w12)

    n_edges = edge_attr.shape[0]
    pack = 128 // DE
    ea8 = edge_attr.reshape(n_edges // pack, pack * DE)
    w3b = jnp.kron(jnp.eye(pack, dtype=jnp.float32), w3)
    b8 = jnp.tile(b_e, (1, pack))
    grid = 8
    blk = ea8.shape[0] // grid

    def c_body(ea_ref, w_ref, b_ref, o_ref):
        o_ref[...] = jnp.dot(ea_ref[...], w_ref[...],
                             preferred_element_type=jnp.float32) + b_ref[...]

    c8 = pl.pallas_call(
        c_body,
        grid=(grid,),
        in_specs=[
            pl.BlockSpec((blk, pack * DE), lambda i: (i, 0)),
            pl.BlockSpec((pack * DE, pack * DE), lambda i: (0, 0)),
            pl.BlockSpec((1, pack * DE), lambda i: (0, 0)),
        ],
        out_specs=pl.BlockSpec((blk, pack * DE), lambda i: (i, 0)),
        out_shape=jax.ShapeDtypeStruct(ea8.shape, jnp.float32),
    )(ea8, w3b, b8)
    return a, b, c8.reshape(n_edges, DE)


def _tc_post(x, sums_p, cnts_p, wn_x, wn_a, b_n, n_pad):
    n = x.shape[0]

    def body(x_ref, s_ref, c_ref, wx_ref, wa_ref, b_ref, o_ref):
        sums = s_ref[pl.ds(0, n)] + s_ref[pl.ds(n_pad, n)]
        cnts = jnp.maximum(c_ref[pl.ds(0, n)] + c_ref[pl.ds(n_pad, n)], 1.0)
        agg = sums / cnts
        o_ref[...] = (
            jnp.dot(x_ref[...], wx_ref[...], preferred_element_type=jnp.float32)
            + jnp.dot(agg, wa_ref[...], preferred_element_type=jnp.float32)
            + b_ref[...]
        )

    return pl.pallas_call(
        body,
        out_shape=jax.ShapeDtypeStruct((x.shape[0], x.shape[1]), jnp.float32),
    )(x, sums_p, cnts_p, wn_x, wn_a, b_n)


def kernel(x, edge_index, edge_attr, W_edge, b_edge, W_node, b_node):
    n_nodes, d_node = x.shape
    n_edges = edge_index.shape[1]

    w12 = W_edge[: 2 * d_node]
    w12 = jnp.concatenate([w12[:d_node], w12[d_node:]], axis=1)
    w3 = W_edge[2 * d_node:]

    a, b, c = _tc_pre(x, edge_attr, w12, w3, b_edge.reshape(1, DE))

    nch = n_edges // (NW * CSZ)
    rpt, n_pad = _acc_pad(n_nodes)
    src = edge_index[0].reshape(NW, nch, CSZ)
    dst = edge_index[1].reshape(NW, nch, CSZ)
    ones = jnp.ones((CSZ, DE), jnp.float32)
    zeros = jnp.zeros((rpt, DE), jnp.float32)

    h_e, sums_p, cnts_p = _sc_edge_kernel(n_nodes, n_edges)(
        a, b, c, src, dst, ones, zeros)

    h_v = _tc_post(
        x,
        sums_p,
        cnts_p,
        W_node[:d_node],
        W_node[d_node:],
        b_node.reshape(1, d_node),
        n_pad,
    )
    return (h_v, edge_index, h_e)
